# Initial kernel scaffold; baseline (speedup 1.0000x reference)
import functools

import jax
import jax.numpy as jnp
from jax import lax
from jax.experimental import pallas as pl
from jax.experimental.pallas import tpu as pltpu
from jax.experimental.pallas import tpu_sc as plsc

_N = 10000
_D = 128
_E = 320000

_NC = 2
_NS = 16
_NW = _NC * _NS
_EPW = _E // _NW
_K = 80
_NBLK = _EPW // _K
_RPS = _N // _NS
_ZR = 125


def _project_body(x_ref, w_ref, b_ref, o_ref):
    o_ref[...] = (
        jnp.dot(x_ref[...], w_ref[...], preferred_element_type=jnp.float32)
        + b_ref[...]
    )


def _project(features, W_proj, b_proj2):
    blk = 1000
    return pl.pallas_call(
        _project_body,
        grid=(_N // blk,),
        in_specs=[
            pl.BlockSpec((blk, _D), lambda i: (i, 0)),
            pl.BlockSpec((_D, _D), lambda i: (0, 0)),
            pl.BlockSpec((1, _D), lambda i: (0, 0)),
        ],
        out_specs=pl.BlockSpec((blk, _D), lambda i: (i, 0)),
        out_shape=jax.ShapeDtypeStruct((_N, _D), jnp.float32),
    )(features, W_proj, b_proj2)


_mesh = plsc.VectorSubcoreMesh(core_axis_name="c", subcore_axis_name="s")


@functools.partial(
    pl.kernel,
    out_type=(
        jax.ShapeDtypeStruct((_N, _D), jnp.float32),
        jax.ShapeDtypeStruct((_N, _D), jnp.float32),
    ),
    mesh=_mesh,
    scratch_types=[
        pltpu.VMEM((_NBLK, _K), jnp.int32),
        pltpu.VMEM((_NBLK, _K), jnp.int32),
        pltpu.VMEM((_NBLK, _K), jnp.float32),
        pltpu.VMEM((_K, _D), jnp.float32),
        pltpu.VMEM((_ZR, _D), jnp.float32),
        pltpu.VMEM_SHARED((_N, _D), jnp.float32),
        pltpu.SemaphoreType.DMA,
    ],
)
def _aggregate(src_hbm, dst_hbm, ew_hbm, h_hbm, out0, out1,
               src_v, dst_v, ew_v, rows_v, zbuf_v, agg_sh, sem):
    c = lax.axis_index("c")
    s = lax.axis_index("s")
    wid = s * _NC + c

    zeros = jnp.zeros((16,), jnp.float32)

    def zrow(r, carry):
        for d in range(_D // 16):
            zbuf_v[r, pl.ds(d * 16, 16)] = zeros
        return carry

    lax.fori_loop(0, _ZR, zrow, 0)
    for t in range(_RPS // _ZR):
        pltpu.sync_copy(zbuf_v, agg_sh.at[pl.ds(s * _RPS + t * _ZR, _ZR)])
    plsc.subcore_barrier()

    pltpu.sync_copy(src_hbm.at[wid], src_v)
    pltpu.sync_copy(dst_hbm.at[wid], dst_v)
    pltpu.sync_copy(ew_hbm.at[wid], ew_v)

    def block(j, carry):
        pltpu.async_copy(h_hbm.at[src_v.at[j]], rows_v, sem).wait()

        def scale(e, cc):
            w = ew_v[j, e]
            for d in range(_D // 16):
                sl = pl.ds(d * 16, 16)
                rows_v[e, sl] = rows_v[e, sl] * w
            return cc

        lax.fori_loop(0, _K, scale, 0)
        pltpu.sync_copy(rows_v, agg_sh.at[dst_v.at[j]], add=True)
        return carry

    lax.fori_loop(0, _NBLK, block, 0)

    plsc.subcore_barrier()

    @pl.when(c == 0)
    def _():
        pltpu.sync_copy(agg_sh.at[pl.ds(s * _RPS, _RPS)],
                        out0.at[pl.ds(s * _RPS, _RPS)])

    @pl.when(c == 1)
    def _():
        pltpu.sync_copy(agg_sh.at[pl.ds(s * _RPS, _RPS)],
                        out1.at[pl.ds(s * _RPS, _RPS)])


def _combine_body(h_ref, a0_ref, a1_ref, w_ref, b_ref, o_ref):
    agg = a0_ref[...] + a1_ref[...]
    acc = jnp.dot(h_ref[...], w_ref[0:_D, :], preferred_element_type=jnp.float32)
    acc = acc + jnp.dot(agg, w_ref[_D:2 * _D, :], preferred_element_type=jnp.float32)
    o_ref[...] = jnp.maximum(acc + b_ref[...], 0.0)


def _combine(h, a0, a1, W_agg, b_agg2):
    blk = 1000
    return pl.pallas_call(
        _combine_body,
        grid=(_N // blk,),
        in_specs=[
            pl.BlockSpec((blk, _D), lambda i: (i, 0)),
            pl.BlockSpec((blk, _D), lambda i: (i, 0)),
            pl.BlockSpec((blk, _D), lambda i: (i, 0)),
            pl.BlockSpec((2 * _D, _D), lambda i: (0, 0)),
            pl.BlockSpec((1, _D), lambda i: (0, 0)),
        ],
        out_specs=pl.BlockSpec((blk, _D), lambda i: (i, 0)),
        out_shape=jax.ShapeDtypeStruct((_N, _D), jnp.float32),
    )(h, a0, a1, W_agg, b_agg2)


def kernel(features, edge_index, edge_weight, W_proj, b_proj, W_agg, b_agg):
    src = edge_index[0].astype(jnp.int32).reshape(_NW, _NBLK, _K)
    dst = edge_index[1].astype(jnp.int32).reshape(_NW, _NBLK, _K)
    ew = edge_weight.reshape(_NW, _NBLK, _K)

    h = _project(features, W_proj, b_proj.reshape(1, _D))
    a0, a1 = _aggregate(src, dst, ew, h)
    return _combine(h, a0, a1, W_agg, b_agg.reshape(1, _D))


# speedup vs baseline: 2.5033x; 2.5033x over previous
import functools

import jax
import jax.numpy as jnp
from jax import lax
from jax.experimental import pallas as pl
from jax.experimental.pallas import tpu as pltpu
from jax.experimental.pallas import tpu_sc as plsc

_N = 10000
_D = 128
_H = _D // 2
_E = 320000

_NC = 2
_NS = 16
_NW = _NC * _NS
_EPW = _E // _NW
_K = 128
_NBLK = -(-_EPW // _K)
_EPAD = _NBLK * _K - _EPW
_CHK = 624
_TAIL = _N - _NS * _CHK
_ZR = 208


def _project_body(x_ref, w_ref, b_ref, o_ref, lo_ref, hi_ref):
    acc = (
        jnp.dot(x_ref[...], w_ref[...], preferred_element_type=jnp.float32)
        + b_ref[...]
    )
    o_ref[...] = acc
    lo_ref[...] = acc[:, 0:_H]
    hi_ref[...] = acc[:, _H:_D]


def _project(features, W_proj, b_proj2):
    blk = 1000
    return pl.pallas_call(
        _project_body,
        grid=(_N // blk,),
        in_specs=[
            pl.BlockSpec((blk, _D), lambda i: (i, 0)),
            pl.BlockSpec((_D, _D), lambda i: (0, 0)),
            pl.BlockSpec((1, _D), lambda i: (0, 0)),
        ],
        out_specs=[
            pl.BlockSpec((blk, _D), lambda i: (i, 0)),
            pl.BlockSpec((blk, _H), lambda i: (i, 0)),
            pl.BlockSpec((blk, _H), lambda i: (i, 0)),
        ],
        out_shape=[
            jax.ShapeDtypeStruct((_N, _D), jnp.float32),
            jax.ShapeDtypeStruct((_N, _H), jnp.float32),
            jax.ShapeDtypeStruct((_N, _H), jnp.float32),
        ],
    )(features, W_proj, b_proj2)


_mesh = plsc.VectorSubcoreMesh(core_axis_name="c", subcore_axis_name="s")


@functools.partial(
    pl.kernel,
    out_type=tuple(
        jax.ShapeDtypeStruct((_N, _H), jnp.float32) for _ in range(4)
    ),
    mesh=_mesh,
    compiler_params=pltpu.CompilerParams(use_tc_tiling_on_sc=False),
    scratch_types=[
        pltpu.VMEM((_NBLK, _K), jnp.int32),
        pltpu.VMEM((_NBLK, _K), jnp.int32),
        pltpu.VMEM((_NBLK, _K), jnp.float32),
        pltpu.VMEM((_K, _H), jnp.float32),
        pltpu.VMEM((_ZR, _H), jnp.float32),
        pltpu.VMEM_SHARED((_N, _H), jnp.float32),
        pltpu.SemaphoreType.DMA,
    ],
)
def _aggregate(src_hbm, dst_hbm, ew_hbm, h0_hbm, h1_hbm,
               out00, out01, out10, out11,
               src_v, dst_v, ew_v, rows_v, zbuf_v, agg_sh, sem):
    c = lax.axis_index("c")
    s = lax.axis_index("s")
    wid = s * _NC + c

    pltpu.sync_copy(src_hbm.at[wid], src_v)
    pltpu.sync_copy(dst_hbm.at[wid], dst_v)
    pltpu.sync_copy(ew_hbm.at[wid], ew_v)

    zeros = jnp.zeros((16,), jnp.float32)

    def zrow(r, carry):
        for d in range(_H // 16):
            zbuf_v[r, pl.ds(d * 16, 16)] = zeros
        return carry

    lax.fori_loop(0, _ZR, zrow, 0)

    for p in range(2):
        h_hbm = h0_hbm if p == 0 else h1_hbm

        for t in range(_CHK // _ZR):
            pltpu.sync_copy(zbuf_v, agg_sh.at[pl.ds(s * _CHK + t * _ZR, _ZR)])

        @pl.when(s == 0)
        def _():
            pltpu.sync_copy(zbuf_v.at[pl.ds(0, _TAIL)],
                            agg_sh.at[pl.ds(_NS * _CHK, _TAIL)])

        plsc.subcore_barrier()

        def block(j, carry):
            pltpu.async_copy(h_hbm.at[src_v.at[j]], rows_v, sem).wait()

            def scale(g, cc):
                w16 = ew_v[j, pl.ds(g * 16, 16)]
                for e in range(16):
                    w = w16[e]
                    row = g * 16 + e
                    for d in range(_H // 16):
                        sl = pl.ds(d * 16, 16)
                        rows_v[row, sl] = rows_v[row, sl] * w
                return cc

            lax.fori_loop(0, _K // 16, scale, 0)
            pltpu.sync_copy(rows_v, agg_sh.at[dst_v.at[j]], add=True)
            return carry

        lax.fori_loop(0, _NBLK, block, 0)

        plsc.subcore_barrier()

        out_c0 = out00 if p == 0 else out01
        out_c1 = out10 if p == 0 else out11

        @pl.when(c == 0)
        def _():
            pltpu.sync_copy(agg_sh.at[pl.ds(s * _CHK, _CHK)],
                            out_c0.at[pl.ds(s * _CHK, _CHK)])

            @pl.when(s == 0)
            def _():
                pltpu.sync_copy(agg_sh.at[pl.ds(_NS * _CHK, _TAIL)],
                                out_c0.at[pl.ds(_NS * _CHK, _TAIL)])

        @pl.when(c == 1)
        def _():
            pltpu.sync_copy(agg_sh.at[pl.ds(s * _CHK, _CHK)],
                            out_c1.at[pl.ds(s * _CHK, _CHK)])

            @pl.when(s == 0)
            def _():
                pltpu.sync_copy(agg_sh.at[pl.ds(_NS * _CHK, _TAIL)],
                                out_c1.at[pl.ds(_NS * _CHK, _TAIL)])

        plsc.subcore_barrier()


def _combine_body(h_ref, a00_ref, a01_ref, a10_ref, a11_ref,
                  w_ref, b_ref, o_ref):
    alo = a00_ref[...] + a10_ref[...]
    ahi = a01_ref[...] + a11_ref[...]
    acc = jnp.dot(h_ref[...], w_ref[0:_D, :], preferred_element_type=jnp.float32)
    acc = acc + jnp.dot(alo, w_ref[_D:_D + _H, :],
                        preferred_element_type=jnp.float32)
    acc = acc + jnp.dot(ahi, w_ref[_D + _H:2 * _D, :],
                        preferred_element_type=jnp.float32)
    o_ref[...] = jnp.maximum(acc + b_ref[...], 0.0)


def _combine(h, a00, a01, a10, a11, W_agg, b_agg2):
    blk = 1000
    return pl.pallas_call(
        _combine_body,
        grid=(_N // blk,),
        in_specs=[
            pl.BlockSpec((blk, _D), lambda i: (i, 0)),
            pl.BlockSpec((blk, _H), lambda i: (i, 0)),
            pl.BlockSpec((blk, _H), lambda i: (i, 0)),
            pl.BlockSpec((blk, _H), lambda i: (i, 0)),
            pl.BlockSpec((blk, _H), lambda i: (i, 0)),
            pl.BlockSpec((2 * _D, _D), lambda i: (0, 0)),
            pl.BlockSpec((1, _D), lambda i: (0, 0)),
        ],
        out_specs=pl.BlockSpec((blk, _D), lambda i: (i, 0)),
        out_shape=jax.ShapeDtypeStruct((_N, _D), jnp.float32),
    )(h, a00, a01, a10, a11, W_agg, b_agg2)


def _pad_edges(x):
    x2 = x.reshape(_NW, _EPW)
    pad = jnp.zeros((_NW, _EPAD), dtype=x.dtype)
    return jnp.concatenate([x2, pad], axis=1).reshape(_NW, _NBLK, _K)


def kernel(features, edge_index, edge_weight, W_proj, b_proj, W_agg, b_agg):
    src = _pad_edges(edge_index[0].astype(jnp.int32))
    dst = _pad_edges(edge_index[1].astype(jnp.int32))
    ew = _pad_edges(edge_weight)

    h, h0, h1 = _project(features, W_proj, b_proj.reshape(1, _D))
    a00, a01, a10, a11 = _aggregate(src, dst, ew, h0, h1)
    return _combine(h, a00, a01, a10, a11, W_agg, b_agg.reshape(1, _D))
